# baseline (speedup 1.0000x reference)
import functools

import jax
import jax.numpy as jnp
from jax.experimental import pallas as pl

_R_FRAC = 0.01
_V_STEPS = 10


def _mmcl_block(x_ref, t_ref, out_ref, *, K, N):
    x = x_ref[...]
    t = t_ref[...]
    col = jax.lax.broadcasted_iota(jnp.int32, x.shape, 1)
    is_t = col == t
    valid_neg = (col < N) & jnp.logical_not(is_t)
    pos = jnp.sum(jnp.where(is_t, x, 0.0), axis=1, keepdims=True)
    xn = jnp.where(valid_neg, x, -jnp.inf)
    mneg = jnp.max(xn, axis=1, keepdims=True)
    m = jnp.maximum(mneg, pos)
    mn = jnp.min(jnp.where(valid_neg, x, jnp.inf), axis=1, keepdims=True)

    K_ = jnp.int32(K)
    lo0 = mn
    hi0 = mneg + jnp.maximum(jnp.abs(mneg) * 9.8e-4, 1e-30)

    def body(i, carry):
        lo, hi = carry
        mid = 0.5 * (lo + hi)
        cnt = jnp.sum((xn >= mid).astype(jnp.int32), axis=1, keepdims=True)
        ok = cnt >= K_
        return jnp.where(ok, mid, lo), jnp.where(ok, hi, mid)

    lo, hi = jax.lax.fori_loop(0, _V_STEPS, body, (lo0, hi0))

    e = jnp.exp(10.0 * (xn - m))
    ge_hi = xn >= hi
    in_b = (xn >= lo) & jnp.logical_not(ge_hi)
    c_hi = jnp.sum(ge_hi.astype(jnp.int32), axis=1, keepdims=True)
    S_hi = jnp.sum(jnp.where(ge_hi, e, 0.0), axis=1, keepdims=True)
    c_b = jnp.sum(in_b.astype(jnp.int32), axis=1, keepdims=True)
    S_b = jnp.sum(jnp.where(in_b, e, 0.0), axis=1, keepdims=True)
    c_b = jnp.maximum(c_b, 1)
    S = (S_hi
         + (K_ - c_hi).astype(jnp.float32) * S_b / c_b.astype(jnp.float32)
         + jnp.exp(10.0 * (pos - m)))
    out_ref[...] = 10.0 * (m - pos) + jnp.log(S)


def _losses_pallas(logits, t2, *, K, N):
    Bs = logits.shape[0]
    R = 16
    return pl.pallas_call(
        functools.partial(_mmcl_block, K=K, N=N),
        grid=(Bs // R,),
        in_specs=[
            pl.BlockSpec((R, N), lambda i: (i, 0)),
            pl.BlockSpec((R, 1), lambda i: (i, 0)),
        ],
        out_specs=pl.BlockSpec((R, 1), lambda i: (i, 0)),
        out_shape=jax.ShapeDtypeStruct((Bs, 1), jnp.float32),
    )(logits, t2)


def kernel(logits, targets):
    B, N = logits.shape
    K = int(_R_FRAC * (N - 1))
    t2 = targets.reshape(B, 1).astype(jnp.int32)
    f = functools.partial(_losses_pallas, K=K, N=N)

    devs = jax.devices()
    ndev = 2 if (len(devs) >= 2 and B % 16 == 0) else 1
    if ndev > 1:
        import numpy as np
        from jax.sharding import Mesh, PartitionSpec as P
        mesh = Mesh(np.asarray(devs[:ndev]), ("b",))
        f = jax.shard_map(f, mesh=mesh,
                          in_specs=(P("b", None), P("b", None)),
                          out_specs=P("b", None), check_vma=False)
    return jnp.mean(f(logits, t2))


# speedup vs baseline: 1.6871x; 1.6871x over previous
import functools

import jax
import jax.numpy as jnp
from jax.experimental import pallas as pl

_R_FRAC = 0.01
_V_STEPS = 10


def _mmcl_block(x_ref, t_ref, out_ref, *, K, N):
    x = x_ref[...]
    t = t_ref[...]
    col = jax.lax.broadcasted_iota(jnp.int32, x.shape, 1)
    is_t = col == t
    valid_neg = (col < N) & jnp.logical_not(is_t)
    pos = jnp.sum(jnp.where(is_t, x, 0.0), axis=1, keepdims=True)
    xn = jnp.where(valid_neg, x, -jnp.inf)
    mneg = jnp.max(xn, axis=1, keepdims=True)
    m = jnp.maximum(mneg, pos)
    mn = jnp.min(jnp.where(valid_neg, x, jnp.inf), axis=1, keepdims=True)

    K_ = jnp.int32(K)
    lo0 = mn
    hi0 = mneg + jnp.maximum(jnp.abs(mneg) * 9.8e-4, 1e-30)

    def body(i, carry):
        lo, hi = carry
        mid = 0.5 * (lo + hi)
        cnt = jnp.sum((xn >= mid).astype(jnp.int32), axis=1, keepdims=True)
        ok = cnt >= K_
        return jnp.where(ok, mid, lo), jnp.where(ok, hi, mid)

    lo, hi = jax.lax.fori_loop(0, _V_STEPS, body, (lo0, hi0))

    e = jnp.exp(10.0 * (xn - m))
    ge_hi = xn >= hi
    in_b = (xn >= lo) & jnp.logical_not(ge_hi)
    c_hi = jnp.sum(ge_hi.astype(jnp.int32), axis=1, keepdims=True)
    S_hi = jnp.sum(jnp.where(ge_hi, e, 0.0), axis=1, keepdims=True)
    c_b = jnp.sum(in_b.astype(jnp.int32), axis=1, keepdims=True)
    S_b = jnp.sum(jnp.where(in_b, e, 0.0), axis=1, keepdims=True)
    c_b = jnp.maximum(c_b, 1)
    S = (S_hi
         + (K_ - c_hi).astype(jnp.float32) * S_b / c_b.astype(jnp.float32)
         + jnp.exp(10.0 * (pos - m)))
    out_ref[...] = 10.0 * (m - pos) + jnp.log(S)


def _losses_pallas(logits, t2, *, K, N):
    Bs = logits.shape[0]
    R = 16
    return pl.pallas_call(
        functools.partial(_mmcl_block, K=K, N=N),
        grid=(Bs // R,),
        in_specs=[
            pl.BlockSpec((R, N), lambda i: (i, 0)),
            pl.BlockSpec((R, 1), lambda i: (i, 0)),
        ],
        out_specs=pl.BlockSpec((R, 1), lambda i: (i, 0)),
        out_shape=jax.ShapeDtypeStruct((Bs, 1), jnp.float32),
    )(logits, t2)


def kernel(logits, targets):
    B, N = logits.shape
    K = int(_R_FRAC * (N - 1))
    t2 = targets.reshape(B, 1).astype(jnp.int32)
    f = functools.partial(_losses_pallas, K=K, N=N)

    devs = jax.devices()
    ndev = 2 if (len(devs) >= 2 and B % 16 == 0) else 1
    if ndev > 1:
        import numpy as np
        from jax.sharding import Mesh, PartitionSpec as P
        mesh = Mesh(np.asarray(devs[:ndev]), ("b",))
        sh = jax.sharding.NamedSharding(mesh, P("b", None))
        logits = jax.lax.with_sharding_constraint(logits, sh)
        t2 = jax.lax.with_sharding_constraint(t2, sh)
        f = jax.shard_map(f, mesh=mesh,
                          in_specs=(P("b", None), P("b", None)),
                          out_specs=P("b", None), check_vma=False)
    return jnp.mean(f(logits, t2))
